# baseline (speedup 1.0000x reference)
import functools

import jax
import jax.numpy as jnp
from jax.experimental import pallas as pl
from jax.experimental.pallas import tpu as pltpu

N = 10000
D = 256
H = 256
RB = 1000
NROW = N // RB


def _fold_body(Wn_ref, Ws_ref, Wu_ref, bn_ref, bs_ref, bu_ref, A_ref, B_ref, c_ref):
    Wu_top = Wu_ref[:H, :]
    Wu_bot = Wu_ref[H:, :]
    A_ref[...] = jnp.dot(Wn_ref[...], Wu_top, preferred_element_type=jnp.float32)
    B_ref[...] = jnp.dot(Ws_ref[...], Wu_bot, preferred_element_type=jnp.float32)
    c_ref[...] = (
        jnp.dot(bn_ref[...], Wu_top, preferred_element_type=jnp.float32)
        + jnp.dot(bs_ref[...], Wu_bot, preferred_element_type=jnp.float32)
        + bu_ref[...]
    )


def _fold(Wn, bn, Ws, bs, Wu, bu):
    return pl.pallas_call(
        _fold_body,
        out_shape=(
            jax.ShapeDtypeStruct((D, H), jnp.float32),
            jax.ShapeDtypeStruct((D, H), jnp.float32),
            jax.ShapeDtypeStruct((1, H), jnp.float32),
        ),
    )(Wn, Ws, Wu, bn.reshape(1, H), bs.reshape(1, H), bu.reshape(1, H))


def _pre_body(x_ref, A_ref, B_ref, c_ref, y_ref, z_ref):
    xb = jax.nn.relu(x_ref[...])
    y_ref[0] = jnp.dot(xb, A_ref[...], preferred_element_type=jnp.float32)
    z_ref[...] = jnp.dot(xb, B_ref[...], preferred_element_type=jnp.float32) + c_ref[...]


def _pre(x, A, B, c):
    return pl.pallas_call(
        _pre_body,
        grid=(NROW, 2),
        in_specs=[
            pl.BlockSpec((RB, D), lambda r, h: (r, 0)),
            pl.BlockSpec((D, H // 2), lambda r, h: (0, h)),
            pl.BlockSpec((D, H // 2), lambda r, h: (0, h)),
            pl.BlockSpec((1, H // 2), lambda r, h: (0, h)),
        ],
        out_specs=[
            pl.BlockSpec((1, RB, H // 2), lambda r, h: (h, r, 0)),
            pl.BlockSpec((RB, H // 2), lambda r, h: (r, h)),
        ],
        out_shape=(
            jax.ShapeDtypeStruct((2, N, H // 2), jnp.float32),
            jax.ShapeDtypeStruct((N, H), jnp.float32),
        ),
    )(x, A, B, c)


def _mid_body(s_ref, cnt_ref, z_ref, A_ref, B_ref, c_ref, y_ref, z2_ref):
    s = jnp.concatenate([s_ref[0], s_ref[1]], axis=-1)
    cinv = 1.0 / jnp.maximum(cnt_ref[...], 1.0)
    x2 = jax.nn.relu(s * cinv + z_ref[...])
    y_ref[0] = jnp.dot(x2, A_ref[...], preferred_element_type=jnp.float32)
    z2_ref[...] = jnp.dot(x2, B_ref[...], preferred_element_type=jnp.float32) + c_ref[...]


def _mid(s_split, cnt, z1, A, B, c):
    return pl.pallas_call(
        _mid_body,
        grid=(NROW, 2),
        in_specs=[
            pl.BlockSpec((2, RB, H // 2), lambda r, h: (0, r, 0)),
            pl.BlockSpec((RB, 1), lambda r, h: (r, 0)),
            pl.BlockSpec((RB, H), lambda r, h: (r, 0)),
            pl.BlockSpec((D, H // 2), lambda r, h: (0, h)),
            pl.BlockSpec((D, H // 2), lambda r, h: (0, h)),
            pl.BlockSpec((1, H // 2), lambda r, h: (0, h)),
        ],
        out_specs=[
            pl.BlockSpec((1, RB, H // 2), lambda r, h: (h, r, 0)),
            pl.BlockSpec((RB, H // 2), lambda r, h: (r, h)),
        ],
        out_shape=(
            jax.ShapeDtypeStruct((2, N, H // 2), jnp.float32),
            jax.ShapeDtypeStruct((N, H), jnp.float32),
        ),
    )(s_split, cnt, z1, A, B, c)


def _post_body(s_ref, cnt_ref, z_ref, x3_ref):
    s = jnp.concatenate([s_ref[0], s_ref[1]], axis=-1)
    cinv = 1.0 / jnp.maximum(cnt_ref[...], 1.0)
    x3_ref[...] = s * cinv + z_ref[...]


def _post(s_split, cnt, z2):
    return pl.pallas_call(
        _post_body,
        grid=(NROW,),
        in_specs=[
            pl.BlockSpec((2, RB, H // 2), lambda r: (0, r, 0)),
            pl.BlockSpec((RB, 1), lambda r: (r, 0)),
            pl.BlockSpec((RB, H), lambda r: (r, 0)),
        ],
        out_specs=pl.BlockSpec((RB, H), lambda r: (r, 0)),
        out_shape=jax.ShapeDtypeStruct((N, H), jnp.float32),
    )(s_split, cnt, z2)


def _segment_mean_split(y_split, src, dst):
    y = jnp.concatenate([y_split[0], y_split[1]], axis=1)
    msgs = jnp.take(y, src, axis=0)
    s = jax.ops.segment_sum(msgs, dst, num_segments=N)
    return jnp.stack([s[:, :128], s[:, 128:]])


def kernel(node_feature_n1, edge_index_n1_e1_n1, edge_label_index_n1_e1_n1,
           W_neigh1, b_neigh1, W_self1, b_self1, W_update1, b_update1,
           W_neigh2, b_neigh2, W_self2, b_self2, W_update2, b_update2):
    src = edge_index_n1_e1_n1[0]
    dst = edge_index_n1_e1_n1[1]
    A1, B1, c1 = _fold(W_neigh1, b_neigh1, W_self1, b_self1, W_update1, b_update1)
    A2, B2, c2 = _fold(W_neigh2, b_neigh2, W_self2, b_self2, W_update2, b_update2)

    y1, z1 = _pre(node_feature_n1, A1, B1, c1)
    s1 = _segment_mean_split(y1, src, dst)
    cnt = jax.ops.segment_sum(jnp.ones((src.shape[0],), jnp.float32), dst,
                              num_segments=N).reshape(N, 1)
    y2, z2 = _mid(s1, cnt, z1, A2, B2, c2)
    s2 = _segment_mean_split(y2, src, dst)
    x3 = _post(s2, cnt, z2)

    la = edge_label_index_n1_e1_n1[0]
    lb = edge_label_index_n1_e1_n1[1]
    pred = jnp.sum(jnp.take(x3, la, axis=0) * jnp.take(x3, lb, axis=0), axis=-1)
    return pred


# speedup vs baseline: 2.8769x; 2.8769x over previous
import functools

import jax
import jax.numpy as jnp
from jax import lax
from jax.experimental import pallas as pl
from jax.experimental.pallas import tpu as pltpu
from jax.experimental.pallas import tpu_sc as plsc

N = 10000
D = 256
H = 256
HH = H // 2
E = 160000
RB = 1000
NROW = N // RB

EPT = 128
NCHUNK = E // EPT
NSUB = 16
CPT = NCHUNK // NSUB
NEXTRA = NCHUNK - CPT * NSUB
NPAD = 10240
RPT = NPAD // NSUB
CPS = RPT


def _fold_body(Wn_ref, Ws_ref, Wu_ref, bn_ref, bs_ref, bu_ref, A_ref, B_ref, c_ref):
    Wu_top = Wu_ref[:H, :]
    Wu_bot = Wu_ref[H:, :]
    A_ref[...] = jnp.dot(Wn_ref[...], Wu_top, preferred_element_type=jnp.float32)
    B_ref[...] = jnp.dot(Ws_ref[...], Wu_bot, preferred_element_type=jnp.float32)
    c_ref[...] = (
        jnp.dot(bn_ref[...], Wu_top, preferred_element_type=jnp.float32)
        + jnp.dot(bs_ref[...], Wu_bot, preferred_element_type=jnp.float32)
        + bu_ref[...]
    )


def _fold(Wn, bn, Ws, bs, Wu, bu):
    return pl.pallas_call(
        _fold_body,
        out_shape=(
            jax.ShapeDtypeStruct((D, H), jnp.float32),
            jax.ShapeDtypeStruct((D, H), jnp.float32),
            jax.ShapeDtypeStruct((1, H), jnp.float32),
        ),
    )(Wn, Ws, Wu, bn.reshape(1, H), bs.reshape(1, H), bu.reshape(1, H))


def _pre_body(x_ref, A_ref, B_ref, c_ref, ylo_ref, yhi_ref, z_ref):
    xb = jax.nn.relu(x_ref[...])
    y = jnp.dot(xb, A_ref[...], preferred_element_type=jnp.float32)
    ylo_ref[...] = y[:, :HH]
    yhi_ref[...] = y[:, HH:]
    z_ref[...] = jnp.dot(xb, B_ref[...], preferred_element_type=jnp.float32) + c_ref[...]


def _pre(x, A, B, c):
    return pl.pallas_call(
        _pre_body,
        grid=(NROW,),
        in_specs=[
            pl.BlockSpec((RB, D), lambda r: (r, 0)),
            pl.BlockSpec((D, H), lambda r: (0, 0)),
            pl.BlockSpec((D, H), lambda r: (0, 0)),
            pl.BlockSpec((1, H), lambda r: (0, 0)),
        ],
        out_specs=[
            pl.BlockSpec((RB, HH), lambda r: (r, 0)),
            pl.BlockSpec((RB, HH), lambda r: (r, 0)),
            pl.BlockSpec((RB, H), lambda r: (r, 0)),
        ],
        out_shape=(
            jax.ShapeDtypeStruct((N, HH), jnp.float32),
            jax.ShapeDtypeStruct((N, HH), jnp.float32),
            jax.ShapeDtypeStruct((N, H), jnp.float32),
        ),
    )(x, A, B, c)


def _mid_body(slo_ref, shi_ref, cnt_ref, z_ref, A_ref, B_ref, c_ref,
              ylo_ref, yhi_ref, z2_ref):
    s = jnp.concatenate([slo_ref[...], shi_ref[...]], axis=-1)
    cinv = 1.0 / jnp.maximum(cnt_ref[...], 1.0)
    x2 = jax.nn.relu(s * cinv + z_ref[...])
    y = jnp.dot(x2, A_ref[...], preferred_element_type=jnp.float32)
    ylo_ref[...] = y[:, :HH]
    yhi_ref[...] = y[:, HH:]
    z2_ref[...] = jnp.dot(x2, B_ref[...], preferred_element_type=jnp.float32) + c_ref[...]


def _mid(slo, shi, cnt, z1, A, B, c):
    return pl.pallas_call(
        _mid_body,
        grid=(NROW,),
        in_specs=[
            pl.BlockSpec((RB, HH), lambda r: (r, 0)),
            pl.BlockSpec((RB, HH), lambda r: (r, 0)),
            pl.BlockSpec((RB, 1), lambda r: (r, 0)),
            pl.BlockSpec((RB, H), lambda r: (r, 0)),
            pl.BlockSpec((D, H), lambda r: (0, 0)),
            pl.BlockSpec((D, H), lambda r: (0, 0)),
            pl.BlockSpec((1, H), lambda r: (0, 0)),
        ],
        out_specs=[
            pl.BlockSpec((RB, HH), lambda r: (r, 0)),
            pl.BlockSpec((RB, HH), lambda r: (r, 0)),
            pl.BlockSpec((RB, H), lambda r: (r, 0)),
        ],
        out_shape=(
            jax.ShapeDtypeStruct((N, HH), jnp.float32),
            jax.ShapeDtypeStruct((N, HH), jnp.float32),
            jax.ShapeDtypeStruct((N, H), jnp.float32),
        ),
    )(slo, shi, cnt, z1, A, B, c)


def _post_body(slo_ref, shi_ref, cnt_ref, z_ref, x3_ref):
    s = jnp.concatenate([slo_ref[...], shi_ref[...]], axis=-1)
    cinv = 1.0 / jnp.maximum(cnt_ref[...], 1.0)
    x3_ref[...] = s * cinv + z_ref[...]


def _post(slo, shi, cnt, z2):
    return pl.pallas_call(
        _post_body,
        grid=(NROW,),
        in_specs=[
            pl.BlockSpec((RB, HH), lambda r: (r, 0)),
            pl.BlockSpec((RB, HH), lambda r: (r, 0)),
            pl.BlockSpec((RB, 1), lambda r: (r, 0)),
            pl.BlockSpec((RB, H), lambda r: (r, 0)),
        ],
        out_specs=pl.BlockSpec((RB, H), lambda r: (r, 0)),
        out_shape=jax.ShapeDtypeStruct((N, H), jnp.float32),
    )(slo, shi, cnt, z2)


def _seg_body(ylo_hbm, yhi_hbm, src_hbm, dst_hbm, zrow_hbm, zcnt_hbm, ones_hbm,
              slo_hbm, shi_hbm, cnt_hbm,
              acc_sh, cnt_sh, src_v, dst_v, rows_v, ones_v, sem):
    c = lax.axis_index("c")
    s = lax.axis_index("s")

    pltpu.sync_copy(zrow_hbm, acc_sh.at[pl.ds(s * RPT, RPT)])

    @pl.when(c == 0)
    def _():
        pltpu.sync_copy(zcnt_hbm, cnt_sh.at[pl.ds(s * CPS, CPS)])
        pltpu.sync_copy(ones_hbm, ones_v)

    plsc.subcore_barrier()

    def do_chunk(gid):
        base = gid * EPT
        pltpu.sync_copy(src_hbm.at[pl.ds(base, EPT)], src_v)
        pltpu.sync_copy(dst_hbm.at[pl.ds(base, EPT)], dst_v)

        @pl.when(c == 0)
        def _():
            pltpu.async_copy(ylo_hbm.at[src_v], rows_v, sem).wait()
            pltpu.sync_copy(ones_v, cnt_sh.at[dst_v], add=True)

        @pl.when(c == 1)
        def _():
            pltpu.async_copy(yhi_hbm.at[src_v], rows_v, sem).wait()

        pltpu.sync_copy(rows_v, acc_sh.at[dst_v], add=True)

    def loop_body(k, carry):
        do_chunk(s * CPT + k)
        return carry

    lax.fori_loop(0, CPT, loop_body, 0)

    @pl.when(s < NEXTRA)
    def _():
        do_chunk(NSUB * CPT + s)

    plsc.subcore_barrier()

    @pl.when(c == 0)
    def _():
        pltpu.sync_copy(acc_sh.at[pl.ds(s * RPT, RPT)], slo_hbm.at[pl.ds(s * RPT, RPT)])
        pltpu.sync_copy(cnt_sh.at[pl.ds(s * CPS, CPS)], cnt_hbm.at[pl.ds(s * CPS, CPS)])

    @pl.when(c == 1)
    def _():
        pltpu.sync_copy(acc_sh.at[pl.ds(s * RPT, RPT)], shi_hbm.at[pl.ds(s * RPT, RPT)])


def _segment_sum_sc(ylo, yhi, src, dst):
    zrow = jnp.zeros((RPT, HH), jnp.float32)
    zcnt = jnp.zeros((CPS,), jnp.float32)
    ones = jnp.ones((EPT,), jnp.float32)
    mesh = plsc.VectorSubcoreMesh(core_axis_name="c", subcore_axis_name="s")
    return pl.kernel(
        _seg_body,
        out_type=(
            jax.ShapeDtypeStruct((NPAD, HH), jnp.float32),
            jax.ShapeDtypeStruct((NPAD, HH), jnp.float32),
            jax.ShapeDtypeStruct((NPAD,), jnp.float32),
        ),
        mesh=mesh,
        scratch_types=[
            pltpu.VMEM_SHARED((NPAD, HH), jnp.float32),
            pltpu.VMEM_SHARED((NPAD,), jnp.float32),
            pltpu.VMEM((EPT,), jnp.int32),
            pltpu.VMEM((EPT,), jnp.int32),
            pltpu.VMEM((EPT, HH), jnp.float32),
            pltpu.VMEM((EPT,), jnp.float32),
            pltpu.SemaphoreType.DMA,
        ],
    )(ylo, yhi, src, dst, zrow, zcnt, ones)


def kernel(node_feature_n1, edge_index_n1_e1_n1, edge_label_index_n1_e1_n1,
           W_neigh1, b_neigh1, W_self1, b_self1, W_update1, b_update1,
           W_neigh2, b_neigh2, W_self2, b_self2, W_update2, b_update2):
    src = edge_index_n1_e1_n1[0]
    dst = edge_index_n1_e1_n1[1]
    A1, B1, c1 = _fold(W_neigh1, b_neigh1, W_self1, b_self1, W_update1, b_update1)
    A2, B2, c2 = _fold(W_neigh2, b_neigh2, W_self2, b_self2, W_update2, b_update2)

    ylo1, yhi1, z1 = _pre(node_feature_n1, A1, B1, c1)
    slo1, shi1, cnt_pad = _segment_sum_sc(ylo1, yhi1, src, dst)
    cnt = cnt_pad[:NPAD].reshape(NPAD, 1)
    ylo2, yhi2, z2 = _mid(slo1, shi1, cnt, z1, A2, B2, c2)
    slo2, shi2, _ = _segment_sum_sc(ylo2, yhi2, src, dst)
    x3 = _post(slo2, shi2, cnt, z2)

    la = edge_label_index_n1_e1_n1[0]
    lb = edge_label_index_n1_e1_n1[1]
    pred = jnp.sum(jnp.take(x3, la, axis=0) * jnp.take(x3, lb, axis=0), axis=-1)
    return pred


# speedup vs baseline: 3.3796x; 1.1747x over previous
import functools

import jax
import jax.numpy as jnp
from jax import lax
from jax.experimental import pallas as pl
from jax.experimental.pallas import tpu as pltpu
from jax.experimental.pallas import tpu_sc as plsc

N = 10000
D = 256
H = 256
HH = H // 2
E = 160000
RB = 1000
NROW = N // RB

EPT = 128
NCHUNK = E // EPT
NSUB = 16
CPT = NCHUNK // NSUB
NEXTRA = NCHUNK - CPT * NSUB
NPAD = 10240
RPT = NPAD // NSUB
CPS = RPT


def _fold_body(Wn_ref, Ws_ref, Wu_ref, bn_ref, bs_ref, bu_ref, A_ref, B_ref, c_ref):
    Wu_top = Wu_ref[:H, :]
    Wu_bot = Wu_ref[H:, :]
    A_ref[...] = jnp.dot(Wn_ref[...], Wu_top, preferred_element_type=jnp.float32)
    B_ref[...] = jnp.dot(Ws_ref[...], Wu_bot, preferred_element_type=jnp.float32)
    c_ref[...] = (
        jnp.dot(bn_ref[...], Wu_top, preferred_element_type=jnp.float32)
        + jnp.dot(bs_ref[...], Wu_bot, preferred_element_type=jnp.float32)
        + bu_ref[...]
    )


def _fold(Wn, bn, Ws, bs, Wu, bu):
    return pl.pallas_call(
        _fold_body,
        out_shape=(
            jax.ShapeDtypeStruct((D, H), jnp.float32),
            jax.ShapeDtypeStruct((D, H), jnp.float32),
            jax.ShapeDtypeStruct((1, H), jnp.float32),
        ),
    )(Wn, Ws, Wu, bn.reshape(1, H), bs.reshape(1, H), bu.reshape(1, H))


def _pre_body(x_ref, A_ref, B_ref, c_ref, ylo_ref, yhi_ref, z_ref):
    xb = jax.nn.relu(x_ref[...])
    y = jnp.dot(xb, A_ref[...], preferred_element_type=jnp.float32)
    ylo_ref[...] = y[:, :HH]
    yhi_ref[...] = y[:, HH:]
    z_ref[...] = jnp.dot(xb, B_ref[...], preferred_element_type=jnp.float32) + c_ref[...]


def _pre(x, A, B, c):
    return pl.pallas_call(
        _pre_body,
        grid=(NROW,),
        in_specs=[
            pl.BlockSpec((RB, D), lambda r: (r, 0)),
            pl.BlockSpec((D, H), lambda r: (0, 0)),
            pl.BlockSpec((D, H), lambda r: (0, 0)),
            pl.BlockSpec((1, H), lambda r: (0, 0)),
        ],
        out_specs=[
            pl.BlockSpec((RB, HH), lambda r: (r, 0)),
            pl.BlockSpec((RB, HH), lambda r: (r, 0)),
            pl.BlockSpec((RB, H), lambda r: (r, 0)),
        ],
        out_shape=(
            jax.ShapeDtypeStruct((N, HH), jnp.float32),
            jax.ShapeDtypeStruct((N, HH), jnp.float32),
            jax.ShapeDtypeStruct((N, H), jnp.float32),
        ),
    )(x, A, B, c)


def _mid_body(slo_ref, shi_ref, cnt_ref, z_ref, A_ref, B_ref, c_ref,
              ylo_ref, yhi_ref, z2_ref):
    s = jnp.concatenate([slo_ref[...], shi_ref[...]], axis=-1)
    cinv = 1.0 / jnp.maximum(cnt_ref[...], 1.0)
    x2 = jax.nn.relu(s * cinv + z_ref[...])
    y = jnp.dot(x2, A_ref[...], preferred_element_type=jnp.float32)
    ylo_ref[...] = y[:, :HH]
    yhi_ref[...] = y[:, HH:]
    z2_ref[...] = jnp.dot(x2, B_ref[...], preferred_element_type=jnp.float32) + c_ref[...]


def _mid(slo, shi, cnt, z1, A, B, c):
    return pl.pallas_call(
        _mid_body,
        grid=(NROW,),
        in_specs=[
            pl.BlockSpec((RB, HH), lambda r: (r, 0)),
            pl.BlockSpec((RB, HH), lambda r: (r, 0)),
            pl.BlockSpec((RB, 1), lambda r: (r, 0)),
            pl.BlockSpec((RB, H), lambda r: (r, 0)),
            pl.BlockSpec((D, H), lambda r: (0, 0)),
            pl.BlockSpec((D, H), lambda r: (0, 0)),
            pl.BlockSpec((1, H), lambda r: (0, 0)),
        ],
        out_specs=[
            pl.BlockSpec((RB, HH), lambda r: (r, 0)),
            pl.BlockSpec((RB, HH), lambda r: (r, 0)),
            pl.BlockSpec((RB, H), lambda r: (r, 0)),
        ],
        out_shape=(
            jax.ShapeDtypeStruct((N, HH), jnp.float32),
            jax.ShapeDtypeStruct((N, HH), jnp.float32),
            jax.ShapeDtypeStruct((N, H), jnp.float32),
        ),
    )(slo, shi, cnt, z1, A, B, c)


def _post_body(slo_ref, shi_ref, cnt_ref, z_ref, x3_ref):
    s = jnp.concatenate([slo_ref[...], shi_ref[...]], axis=-1)
    cinv = 1.0 / jnp.maximum(cnt_ref[...], 1.0)
    x3_ref[...] = s * cinv + z_ref[...]


def _post(slo, shi, cnt, z2):
    return pl.pallas_call(
        _post_body,
        grid=(NROW,),
        in_specs=[
            pl.BlockSpec((RB, HH), lambda r: (r, 0)),
            pl.BlockSpec((RB, HH), lambda r: (r, 0)),
            pl.BlockSpec((RB, 1), lambda r: (r, 0)),
            pl.BlockSpec((RB, H), lambda r: (r, 0)),
        ],
        out_specs=pl.BlockSpec((RB, H), lambda r: (r, 0)),
        out_shape=jax.ShapeDtypeStruct((N, H), jnp.float32),
    )(slo, shi, cnt, z2)


def _seg_body(ylo_hbm, yhi_hbm, src_hbm, dst_hbm, zrow_hbm, zcnt_hbm, ones_hbm,
              slo_hbm, shi_hbm, cnt_hbm,
              acc_sh, cnt_sh, src_v, dst_v, rows_v, ones_v, sem):
    c = lax.axis_index("c")
    s = lax.axis_index("s")

    pltpu.sync_copy(zrow_hbm, acc_sh.at[pl.ds(s * RPT, RPT)])

    @pl.when(c == 0)
    def _():
        pltpu.sync_copy(zcnt_hbm, cnt_sh.at[pl.ds(s * CPS, CPS)])
        pltpu.sync_copy(ones_hbm, ones_v)

    plsc.subcore_barrier()

    def do_chunk(gid):
        base = gid * EPT
        pltpu.sync_copy(src_hbm.at[pl.ds(base, EPT)], src_v)
        pltpu.sync_copy(dst_hbm.at[pl.ds(base, EPT)], dst_v)

        @pl.when(c == 0)
        def _():
            pltpu.async_copy(ylo_hbm.at[src_v], rows_v, sem).wait()
            pltpu.sync_copy(ones_v, cnt_sh.at[dst_v], add=True)

        @pl.when(c == 1)
        def _():
            pltpu.async_copy(yhi_hbm.at[src_v], rows_v, sem).wait()

        pltpu.sync_copy(rows_v, acc_sh.at[dst_v], add=True)

    def loop_body(k, carry):
        do_chunk(s * CPT + k)
        return carry

    lax.fori_loop(0, CPT, loop_body, 0)

    @pl.when(s < NEXTRA)
    def _():
        do_chunk(NSUB * CPT + s)

    plsc.subcore_barrier()

    @pl.when(c == 0)
    def _():
        pltpu.sync_copy(acc_sh.at[pl.ds(s * RPT, RPT)], slo_hbm.at[pl.ds(s * RPT, RPT)])
        pltpu.sync_copy(cnt_sh.at[pl.ds(s * CPS, CPS)], cnt_hbm.at[pl.ds(s * CPS, CPS)])

    @pl.when(c == 1)
    def _():
        pltpu.sync_copy(acc_sh.at[pl.ds(s * RPT, RPT)], shi_hbm.at[pl.ds(s * RPT, RPT)])


def _segment_sum_sc(ylo, yhi, src, dst):
    zrow = jnp.zeros((RPT, HH), jnp.float32)
    zcnt = jnp.zeros((CPS,), jnp.float32)
    ones = jnp.ones((EPT,), jnp.float32)
    mesh = plsc.VectorSubcoreMesh(core_axis_name="c", subcore_axis_name="s")
    return pl.kernel(
        _seg_body,
        out_type=(
            jax.ShapeDtypeStruct((NPAD, HH), jnp.float32),
            jax.ShapeDtypeStruct((NPAD, HH), jnp.float32),
            jax.ShapeDtypeStruct((NPAD,), jnp.float32),
        ),
        mesh=mesh,
        scratch_types=[
            pltpu.VMEM_SHARED((NPAD, HH), jnp.float32),
            pltpu.VMEM_SHARED((NPAD,), jnp.float32),
            pltpu.VMEM((EPT,), jnp.int32),
            pltpu.VMEM((EPT,), jnp.int32),
            pltpu.VMEM((EPT, HH), jnp.float32),
            pltpu.VMEM((EPT,), jnp.float32),
            pltpu.SemaphoreType.DMA,
        ],
    )(ylo, yhi, src, dst, zrow, zcnt, ones)


NLBL = 100000
LPAD = 100352
NW = 32
LPW = LPAD // NW
LEPT = 112
LCH = LPW // LEPT


def _label_body(x3_hbm, la_hbm, lb_hbm, pred_hbm,
                ia_v, ib_v, rA, rB, out_v, sem):
    c = lax.axis_index("c")
    s = lax.axis_index("s")
    wid = s * 2 + c
    base_t = wid * LPW

    @pl.loop(0, LCH)
    def _chunk(k):
        base = base_t + k * LEPT
        pltpu.sync_copy(la_hbm.at[pl.ds(base, LEPT)], ia_v)
        pltpu.sync_copy(lb_hbm.at[pl.ds(base, LEPT)], ib_v)
        d1 = pltpu.async_copy(x3_hbm.at[ia_v], rA, sem)
        d2 = pltpu.async_copy(x3_hbm.at[ib_v], rB, sem)
        d1.wait()
        d2.wait()

        @pl.loop(0, LEPT // 16)
        def _group(g):
            lane = lax.iota(jnp.int32, 16)
            res = jnp.zeros((16,), jnp.float32)
            for j in range(16):
                e = g * 16 + j
                acc = rA[e, pl.ds(0, 16)] * rB[e, pl.ds(0, 16)]
                for t in range(1, 16):
                    acc = acc + rA[e, pl.ds(16 * t, 16)] * rB[e, pl.ds(16 * t, 16)]
                for sh in (8, 4, 2, 1):
                    acc = acc + acc.at[lane ^ sh].get(mode="promise_in_bounds")
                res = jnp.where(lane == j, acc, res)
            out_v[pl.ds(g * 16, 16)] = res

        pltpu.sync_copy(out_v, pred_hbm.at[pl.ds(base, LEPT)])


def _label_dot_sc(x3, la, lb):
    mesh = plsc.VectorSubcoreMesh(core_axis_name="c", subcore_axis_name="s")
    la_p = jnp.zeros((LPAD,), la.dtype).at[:NLBL].set(la)
    lb_p = jnp.zeros((LPAD,), lb.dtype).at[:NLBL].set(lb)
    pred = pl.kernel(
        _label_body,
        out_type=jax.ShapeDtypeStruct((LPAD,), jnp.float32),
        mesh=mesh,
        scratch_types=[
            pltpu.VMEM((LEPT,), jnp.int32),
            pltpu.VMEM((LEPT,), jnp.int32),
            pltpu.VMEM((LEPT, H), jnp.float32),
            pltpu.VMEM((LEPT, H), jnp.float32),
            pltpu.VMEM((LEPT,), jnp.float32),
            pltpu.SemaphoreType.DMA,
        ],
    )(x3, la_p, lb_p)
    return pred[:NLBL]


def kernel(node_feature_n1, edge_index_n1_e1_n1, edge_label_index_n1_e1_n1,
           W_neigh1, b_neigh1, W_self1, b_self1, W_update1, b_update1,
           W_neigh2, b_neigh2, W_self2, b_self2, W_update2, b_update2):
    src = edge_index_n1_e1_n1[0]
    dst = edge_index_n1_e1_n1[1]
    A1, B1, c1 = _fold(W_neigh1, b_neigh1, W_self1, b_self1, W_update1, b_update1)
    A2, B2, c2 = _fold(W_neigh2, b_neigh2, W_self2, b_self2, W_update2, b_update2)

    ylo1, yhi1, z1 = _pre(node_feature_n1, A1, B1, c1)
    slo1, shi1, cnt_pad = _segment_sum_sc(ylo1, yhi1, src, dst)
    cnt = cnt_pad[:NPAD].reshape(NPAD, 1)
    ylo2, yhi2, z2 = _mid(slo1, shi1, cnt, z1, A2, B2, c2)
    slo2, shi2, _ = _segment_sum_sc(ylo2, yhi2, src, dst)
    x3 = _post(slo2, shi2, cnt, z2)

    la = edge_label_index_n1_e1_n1[0]
    lb = edge_label_index_n1_e1_n1[1]
    return _label_dot_sc(x3, la, lb)


# speedup vs baseline: 3.7502x; 1.1097x over previous
import functools

import jax
import jax.numpy as jnp
from jax import lax
from jax.experimental import pallas as pl
from jax.experimental.pallas import tpu as pltpu
from jax.experimental.pallas import tpu_sc as plsc

N = 10000
D = 256
H = 256
HH = H // 2
E = 160000
RB = 1000
NROW = N // RB

EPT = 128
NSUB = 16
CPT = 80
NCHUNK = NSUB * CPT
EPAD = NCHUNK * EPT
PADROW = 10016
PCH = CPT // 2
NPAD = 10240
RPT = NPAD // NSUB
CPS = RPT


def _fold_body(Wn_ref, Ws_ref, Wu_ref, bn_ref, bs_ref, bu_ref, A_ref, B_ref, c_ref):
    Wu_top = Wu_ref[:H, :]
    Wu_bot = Wu_ref[H:, :]
    A_ref[...] = jnp.dot(Wn_ref[...], Wu_top, preferred_element_type=jnp.float32)
    B_ref[...] = jnp.dot(Ws_ref[...], Wu_bot, preferred_element_type=jnp.float32)
    c_ref[...] = (
        jnp.dot(bn_ref[...], Wu_top, preferred_element_type=jnp.float32)
        + jnp.dot(bs_ref[...], Wu_bot, preferred_element_type=jnp.float32)
        + bu_ref[...]
    )


def _fold(Wn, bn, Ws, bs, Wu, bu):
    return pl.pallas_call(
        _fold_body,
        out_shape=(
            jax.ShapeDtypeStruct((D, H), jnp.float32),
            jax.ShapeDtypeStruct((D, H), jnp.float32),
            jax.ShapeDtypeStruct((1, H), jnp.float32),
        ),
    )(Wn, Ws, Wu, bn.reshape(1, H), bs.reshape(1, H), bu.reshape(1, H))


def _pre_body(x_ref, A_ref, B_ref, c_ref, ylo_ref, yhi_ref, z_ref):
    xb = jax.nn.relu(x_ref[...])
    y = jnp.dot(xb, A_ref[...], preferred_element_type=jnp.float32)
    ylo_ref[...] = y[:, :HH]
    yhi_ref[...] = y[:, HH:]
    z_ref[...] = jnp.dot(xb, B_ref[...], preferred_element_type=jnp.float32) + c_ref[...]


def _pre(x, A, B, c):
    return pl.pallas_call(
        _pre_body,
        grid=(NROW,),
        in_specs=[
            pl.BlockSpec((RB, D), lambda r: (r, 0)),
            pl.BlockSpec((D, H), lambda r: (0, 0)),
            pl.BlockSpec((D, H), lambda r: (0, 0)),
            pl.BlockSpec((1, H), lambda r: (0, 0)),
        ],
        out_specs=[
            pl.BlockSpec((RB, HH), lambda r: (r, 0)),
            pl.BlockSpec((RB, HH), lambda r: (r, 0)),
            pl.BlockSpec((RB, H), lambda r: (r, 0)),
        ],
        out_shape=(
            jax.ShapeDtypeStruct((N, HH), jnp.float32),
            jax.ShapeDtypeStruct((N, HH), jnp.float32),
            jax.ShapeDtypeStruct((N, H), jnp.float32),
        ),
    )(x, A, B, c)


def _mid_body(slo_ref, shi_ref, cnt_ref, z_ref, A_ref, B_ref, c_ref,
              ylo_ref, yhi_ref, z2_ref):
    s = jnp.concatenate([slo_ref[...], shi_ref[...]], axis=-1)
    cinv = 1.0 / jnp.maximum(cnt_ref[...], 1.0)
    x2 = jax.nn.relu(s * cinv + z_ref[...])
    y = jnp.dot(x2, A_ref[...], preferred_element_type=jnp.float32)
    ylo_ref[...] = y[:, :HH]
    yhi_ref[...] = y[:, HH:]
    z2_ref[...] = jnp.dot(x2, B_ref[...], preferred_element_type=jnp.float32) + c_ref[...]


def _mid(slo, shi, cnt, z1, A, B, c):
    return pl.pallas_call(
        _mid_body,
        grid=(NROW,),
        in_specs=[
            pl.BlockSpec((RB, HH), lambda r: (r, 0)),
            pl.BlockSpec((RB, HH), lambda r: (r, 0)),
            pl.BlockSpec((RB, 1), lambda r: (r, 0)),
            pl.BlockSpec((RB, H), lambda r: (r, 0)),
            pl.BlockSpec((D, H), lambda r: (0, 0)),
            pl.BlockSpec((D, H), lambda r: (0, 0)),
            pl.BlockSpec((1, H), lambda r: (0, 0)),
        ],
        out_specs=[
            pl.BlockSpec((RB, HH), lambda r: (r, 0)),
            pl.BlockSpec((RB, HH), lambda r: (r, 0)),
            pl.BlockSpec((RB, H), lambda r: (r, 0)),
        ],
        out_shape=(
            jax.ShapeDtypeStruct((N, HH), jnp.float32),
            jax.ShapeDtypeStruct((N, HH), jnp.float32),
            jax.ShapeDtypeStruct((N, H), jnp.float32),
        ),
    )(slo, shi, cnt, z1, A, B, c)


def _post_body(slo_ref, shi_ref, cnt_ref, z_ref, x3lo_ref, x3hi_ref):
    s = jnp.concatenate([slo_ref[...], shi_ref[...]], axis=-1)
    cinv = 1.0 / jnp.maximum(cnt_ref[...], 1.0)
    x3 = s * cinv + z_ref[...]
    x3lo_ref[...] = x3[:, :HH]
    x3hi_ref[...] = x3[:, HH:]


def _post(slo, shi, cnt, z2):
    return pl.pallas_call(
        _post_body,
        grid=(NROW,),
        in_specs=[
            pl.BlockSpec((RB, HH), lambda r: (r, 0)),
            pl.BlockSpec((RB, HH), lambda r: (r, 0)),
            pl.BlockSpec((RB, 1), lambda r: (r, 0)),
            pl.BlockSpec((RB, H), lambda r: (r, 0)),
        ],
        out_specs=[
            pl.BlockSpec((RB, HH), lambda r: (r, 0)),
            pl.BlockSpec((RB, HH), lambda r: (r, 0)),
        ],
        out_shape=(
            jax.ShapeDtypeStruct((N, HH), jnp.float32),
            jax.ShapeDtypeStruct((N, HH), jnp.float32),
        ),
    )(slo, shi, cnt, z2)


def _seg_body(ylo_hbm, yhi_hbm, src2_hbm, dst2_hbm, zrow_hbm, zcnt_hbm, ones_hbm,
              slo_hbm, shi_hbm, cnt_hbm,
              acc_sh, cnt_sh, srcs_v, dsts_v, rows0_v, rows1_v, ones_v,
              sem0, sem1):
    c = lax.axis_index("c")
    s = lax.axis_index("s")

    pltpu.sync_copy(zrow_hbm, acc_sh.at[pl.ds(s * RPT, RPT)])

    @pl.when(c == 0)
    def _():
        pltpu.sync_copy(zcnt_hbm, cnt_sh.at[pl.ds(s * CPS, CPS)])
        pltpu.sync_copy(ones_hbm, ones_v)

    plsc.subcore_barrier()

    def start_gather(n, rows_v, sem):
        idx = srcs_v.at[n]

        @pl.when(c == 0)
        def _():
            pltpu.async_copy(ylo_hbm.at[idx], rows_v, sem)

        @pl.when(c == 1)
        def _():
            pltpu.async_copy(yhi_hbm.at[idx], rows_v, sem)

    def wait_gather(n, rows_v, sem):
        idx = srcs_v.at[n]

        @pl.when(c == 0)
        def _():
            pltpu.make_async_copy(ylo_hbm.at[idx], rows_v, sem).wait()

        @pl.when(c == 1)
        def _():
            pltpu.make_async_copy(yhi_hbm.at[idx], rows_v, sem).wait()

    def scatter(n, rows_v):
        idx = dsts_v.at[n]
        pltpu.sync_copy(rows_v, acc_sh.at[idx], add=True)

        @pl.when(c == 0)
        def _():
            pltpu.sync_copy(ones_v, cnt_sh.at[idx], add=True)

    for ph in range(CPT // PCH):
        off = s * CPT + ph * PCH
        pltpu.sync_copy(src2_hbm.at[pl.ds(off, PCH)], srcs_v)
        pltpu.sync_copy(dst2_hbm.at[pl.ds(off, PCH)], dsts_v)
        start_gather(0, rows0_v, sem0)

        @pl.loop(0, PCH // 2)
        def _pair(k):
            n0 = 2 * k
            start_gather(n0 + 1, rows1_v, sem1)
            wait_gather(n0, rows0_v, sem0)
            scatter(n0, rows0_v)

            @pl.when(k < PCH // 2 - 1)
            def _():
                start_gather(n0 + 2, rows0_v, sem0)

            wait_gather(n0 + 1, rows1_v, sem1)
            scatter(n0 + 1, rows1_v)

    plsc.subcore_barrier()

    @pl.when(c == 0)
    def _():
        pltpu.sync_copy(acc_sh.at[pl.ds(s * RPT, RPT)], slo_hbm.at[pl.ds(s * RPT, RPT)])
        pltpu.sync_copy(cnt_sh.at[pl.ds(s * CPS, CPS)], cnt_hbm.at[pl.ds(s * CPS, CPS)])

    @pl.when(c == 1)
    def _():
        pltpu.sync_copy(acc_sh.at[pl.ds(s * RPT, RPT)], shi_hbm.at[pl.ds(s * RPT, RPT)])


def _segment_sum_sc(ylo, yhi, src2, dst2):
    zrow = jnp.zeros((RPT, HH), jnp.float32)
    zcnt = jnp.zeros((CPS,), jnp.float32)
    ones = jnp.ones((EPT,), jnp.float32)
    mesh = plsc.VectorSubcoreMesh(core_axis_name="c", subcore_axis_name="s")
    return pl.kernel(
        _seg_body,
        out_type=(
            jax.ShapeDtypeStruct((NPAD, HH), jnp.float32),
            jax.ShapeDtypeStruct((NPAD, HH), jnp.float32),
            jax.ShapeDtypeStruct((NPAD,), jnp.float32),
        ),
        mesh=mesh,
        scratch_types=[
            pltpu.VMEM_SHARED((NPAD, HH), jnp.float32),
            pltpu.VMEM_SHARED((NPAD,), jnp.float32),
            pltpu.VMEM((PCH, EPT), jnp.int32),
            pltpu.VMEM((PCH, EPT), jnp.int32),
            pltpu.VMEM((EPT, HH), jnp.float32),
            pltpu.VMEM((EPT, HH), jnp.float32),
            pltpu.VMEM((EPT,), jnp.float32),
            pltpu.SemaphoreType.DMA,
            pltpu.SemaphoreType.DMA,
        ],
    )(ylo, yhi, src2, dst2, zrow, zcnt, ones)


NLBL = 100000
LPAD = 100352
NW = 32
LPW = LPAD // NW
LEPT = 112
LCH = LPW // LEPT


def _label_body(x3lo_hbm, x3hi_hbm, la_hbm, lb_hbm, pred_hbm,
                ia_v, ib_v, rA0, rB0, rA1, rB1, out_v, accb_v,
                semA0, semB0, semA1, semB1):
    c = lax.axis_index("c")
    s = lax.axis_index("s")
    wid = s * 2 + c
    base_t = wid * LPW

    pltpu.sync_copy(la_hbm.at[pl.ds(base_t, LPW)], ia_v)
    pltpu.sync_copy(lb_hbm.at[pl.ds(base_t, LPW)], ib_v)

    def start(n, h, rA, rB, sA, sB):
        tbl = x3lo_hbm if h == 0 else x3hi_hbm
        pltpu.async_copy(tbl.at[ia_v.at[pl.ds(n * LEPT, LEPT)]], rA, sA)
        pltpu.async_copy(tbl.at[ib_v.at[pl.ds(n * LEPT, LEPT)]], rB, sB)

    def wait(n, h, rA, rB, sA, sB):
        tbl = x3lo_hbm if h == 0 else x3hi_hbm
        pltpu.make_async_copy(tbl.at[ia_v.at[pl.ds(n * LEPT, LEPT)]], rA, sA).wait()
        pltpu.make_async_copy(tbl.at[ib_v.at[pl.ds(n * LEPT, LEPT)]], rB, sB).wait()

    def compute(n, h, rA, rB):
        @pl.loop(0, LEPT // 16)
        def _group(g):
            lane = lax.iota(jnp.int32, 16)
            res = jnp.zeros((16,), jnp.float32)
            for j in range(16):
                e = g * 16 + j
                acc = rA[e, pl.ds(0, 16)] * rB[e, pl.ds(0, 16)]
                for t in range(1, HH // 16):
                    acc = acc + rA[e, pl.ds(16 * t, 16)] * rB[e, pl.ds(16 * t, 16)]
                if h == 0:
                    accb_v[e, :] = acc
                else:
                    acc = acc + accb_v[e, :]
                    for sh in (8, 4, 2, 1):
                        acc = acc + acc.at[lane ^ sh].get(mode="promise_in_bounds")
                    res = jnp.where(lane == j, acc, res)
            if h == 1:
                out_v[pl.ds(n * LEPT + g * 16, 16)] = res

    start(0, 0, rA0, rB0, semA0, semB0)

    @pl.loop(0, LCH)
    def _chunk(k):
        start(k, 1, rA1, rB1, semA1, semB1)
        wait(k, 0, rA0, rB0, semA0, semB0)
        compute(k, 0, rA0, rB0)

        @pl.when(k < LCH - 1)
        def _():
            start(k + 1, 0, rA0, rB0, semA0, semB0)

        wait(k, 1, rA1, rB1, semA1, semB1)
        compute(k, 1, rA1, rB1)

    pltpu.sync_copy(out_v, pred_hbm.at[pl.ds(base_t, LPW)])


def _label_dot_sc(x3lo, x3hi, la, lb):
    mesh = plsc.VectorSubcoreMesh(core_axis_name="c", subcore_axis_name="s")
    la_p = jnp.zeros((LPAD,), la.dtype).at[:NLBL].set(la)
    lb_p = jnp.zeros((LPAD,), lb.dtype).at[:NLBL].set(lb)
    pred = pl.kernel(
        _label_body,
        out_type=jax.ShapeDtypeStruct((LPAD,), jnp.float32),
        mesh=mesh,
        scratch_types=[
            pltpu.VMEM((LPW,), jnp.int32),
            pltpu.VMEM((LPW,), jnp.int32),
            pltpu.VMEM((LEPT, HH), jnp.float32),
            pltpu.VMEM((LEPT, HH), jnp.float32),
            pltpu.VMEM((LEPT, HH), jnp.float32),
            pltpu.VMEM((LEPT, HH), jnp.float32),
            pltpu.VMEM((LPW,), jnp.float32),
            pltpu.VMEM((LEPT, 16), jnp.float32),
            pltpu.SemaphoreType.DMA,
            pltpu.SemaphoreType.DMA,
            pltpu.SemaphoreType.DMA,
            pltpu.SemaphoreType.DMA,
        ],
    )(x3lo, x3hi, la_p, lb_p)
    return pred[:NLBL]


def kernel(node_feature_n1, edge_index_n1_e1_n1, edge_label_index_n1_e1_n1,
           W_neigh1, b_neigh1, W_self1, b_self1, W_update1, b_update1,
           W_neigh2, b_neigh2, W_self2, b_self2, W_update2, b_update2):
    src = edge_index_n1_e1_n1[0].astype(jnp.int32)
    dst = edge_index_n1_e1_n1[1].astype(jnp.int32)
    src2 = jnp.zeros((EPAD,), jnp.int32).at[:E].set(src).reshape(NCHUNK, EPT)
    dst2 = jnp.full((EPAD,), PADROW, jnp.int32).at[:E].set(dst).reshape(NCHUNK, EPT)
    A1, B1, c1 = _fold(W_neigh1, b_neigh1, W_self1, b_self1, W_update1, b_update1)
    A2, B2, c2 = _fold(W_neigh2, b_neigh2, W_self2, b_self2, W_update2, b_update2)

    ylo1, yhi1, z1 = _pre(node_feature_n1, A1, B1, c1)
    slo1, shi1, cnt_pad = _segment_sum_sc(ylo1, yhi1, src2, dst2)
    cnt = cnt_pad.reshape(NPAD, 1)
    ylo2, yhi2, z2 = _mid(slo1, shi1, cnt, z1, A2, B2, c2)
    slo2, shi2, _ = _segment_sum_sc(ylo2, yhi2, src2, dst2)
    x3lo, x3hi = _post(slo2, shi2, cnt, z2)

    la = edge_label_index_n1_e1_n1[0]
    lb = edge_label_index_n1_e1_n1[1]
    return _label_dot_sc(x3lo, x3hi, la, lb)


# speedup vs baseline: 3.7553x; 1.0014x over previous
import functools

import jax
import jax.numpy as jnp
from jax import lax
from jax.experimental import pallas as pl
from jax.experimental.pallas import tpu as pltpu
from jax.experimental.pallas import tpu_sc as plsc

N = 10000
D = 256
H = 256
HH = H // 2
E = 160000
RB = 1000
NROW = N // RB

EPT = 128
NSUB = 16
CPT = 80
NCHUNK = NSUB * CPT
EPAD = NCHUNK * EPT
PADROW = 10016
PCH = CPT // 2
NPAD = 10240
RPT = NPAD // NSUB
CPS = RPT


def _fold_body(Wn_ref, Ws_ref, Wu_ref, bn_ref, bs_ref, bu_ref, A_ref, B_ref, c_ref):
    Wu_top = Wu_ref[:H, :]
    Wu_bot = Wu_ref[H:, :]
    A_ref[...] = jnp.dot(Wn_ref[...], Wu_top, preferred_element_type=jnp.float32)
    B_ref[...] = jnp.dot(Ws_ref[...], Wu_bot, preferred_element_type=jnp.float32)
    c_ref[...] = (
        jnp.dot(bn_ref[...], Wu_top, preferred_element_type=jnp.float32)
        + jnp.dot(bs_ref[...], Wu_bot, preferred_element_type=jnp.float32)
        + bu_ref[...]
    )


def _fold(Wn, bn, Ws, bs, Wu, bu):
    return pl.pallas_call(
        _fold_body,
        out_shape=(
            jax.ShapeDtypeStruct((D, H), jnp.float32),
            jax.ShapeDtypeStruct((D, H), jnp.float32),
            jax.ShapeDtypeStruct((1, H), jnp.float32),
        ),
    )(Wn, Ws, Wu, bn.reshape(1, H), bs.reshape(1, H), bu.reshape(1, H))


def _pre_body(x_ref, A_ref, B_ref, c_ref, ylo_ref, yhi_ref, z_ref):
    xb = jax.nn.relu(x_ref[...])
    y = jnp.dot(xb, A_ref[...], preferred_element_type=jnp.float32)
    ylo_ref[...] = y[:, :HH]
    yhi_ref[...] = y[:, HH:]
    z_ref[...] = jnp.dot(xb, B_ref[...], preferred_element_type=jnp.float32) + c_ref[...]


def _pre(x, A, B, c):
    return pl.pallas_call(
        _pre_body,
        grid=(NROW,),
        in_specs=[
            pl.BlockSpec((RB, D), lambda r: (r, 0)),
            pl.BlockSpec((D, H), lambda r: (0, 0)),
            pl.BlockSpec((D, H), lambda r: (0, 0)),
            pl.BlockSpec((1, H), lambda r: (0, 0)),
        ],
        out_specs=[
            pl.BlockSpec((RB, HH), lambda r: (r, 0)),
            pl.BlockSpec((RB, HH), lambda r: (r, 0)),
            pl.BlockSpec((RB, H), lambda r: (r, 0)),
        ],
        out_shape=(
            jax.ShapeDtypeStruct((N, HH), jnp.float32),
            jax.ShapeDtypeStruct((N, HH), jnp.float32),
            jax.ShapeDtypeStruct((N, H), jnp.float32),
        ),
    )(x, A, B, c)


def _mid_body(slo_ref, shi_ref, cnt_ref, z_ref, A_ref, B_ref, c_ref,
              ylo_ref, yhi_ref, z2_ref):
    s = jnp.concatenate([slo_ref[...], shi_ref[...]], axis=-1)
    cinv = 1.0 / jnp.maximum(cnt_ref[...], 1.0)
    x2 = jax.nn.relu(s * cinv + z_ref[...])
    y = jnp.dot(x2, A_ref[...], preferred_element_type=jnp.float32)
    ylo_ref[...] = y[:, :HH]
    yhi_ref[...] = y[:, HH:]
    z2_ref[...] = jnp.dot(x2, B_ref[...], preferred_element_type=jnp.float32) + c_ref[...]


def _mid(slo, shi, cnt, z1, A, B, c):
    return pl.pallas_call(
        _mid_body,
        grid=(NROW,),
        in_specs=[
            pl.BlockSpec((RB, HH), lambda r: (r, 0)),
            pl.BlockSpec((RB, HH), lambda r: (r, 0)),
            pl.BlockSpec((RB, 1), lambda r: (r, 0)),
            pl.BlockSpec((RB, H), lambda r: (r, 0)),
            pl.BlockSpec((D, H), lambda r: (0, 0)),
            pl.BlockSpec((D, H), lambda r: (0, 0)),
            pl.BlockSpec((1, H), lambda r: (0, 0)),
        ],
        out_specs=[
            pl.BlockSpec((RB, HH), lambda r: (r, 0)),
            pl.BlockSpec((RB, HH), lambda r: (r, 0)),
            pl.BlockSpec((RB, H), lambda r: (r, 0)),
        ],
        out_shape=(
            jax.ShapeDtypeStruct((N, HH), jnp.float32),
            jax.ShapeDtypeStruct((N, HH), jnp.float32),
            jax.ShapeDtypeStruct((N, H), jnp.float32),
        ),
    )(slo, shi, cnt, z1, A, B, c)


def _post_body(slo_ref, shi_ref, cnt_ref, z_ref, x3lo_ref, x3hi_ref):
    s = jnp.concatenate([slo_ref[...], shi_ref[...]], axis=-1)
    cinv = 1.0 / jnp.maximum(cnt_ref[...], 1.0)
    x3 = s * cinv + z_ref[...]
    x3lo_ref[...] = x3[:, :HH]
    x3hi_ref[...] = x3[:, HH:]


def _post(slo, shi, cnt, z2):
    return pl.pallas_call(
        _post_body,
        grid=(NROW,),
        in_specs=[
            pl.BlockSpec((RB, HH), lambda r: (r, 0)),
            pl.BlockSpec((RB, HH), lambda r: (r, 0)),
            pl.BlockSpec((RB, 1), lambda r: (r, 0)),
            pl.BlockSpec((RB, H), lambda r: (r, 0)),
        ],
        out_specs=[
            pl.BlockSpec((RB, HH), lambda r: (r, 0)),
            pl.BlockSpec((RB, HH), lambda r: (r, 0)),
        ],
        out_shape=(
            jax.ShapeDtypeStruct((N, HH), jnp.float32),
            jax.ShapeDtypeStruct((N, HH), jnp.float32),
        ),
    )(slo, shi, cnt, z2)


def _seg_body(want_cnt, ylo_hbm, yhi_hbm, src2_hbm, dst2_hbm, zrow_hbm, zcnt_hbm,
              ones_hbm, slo_hbm, shi_hbm, cnt_hbm,
              acc_sh, cnt_sh, srcs_v, dsts_v, rows0_v, rows1_v, ones_v,
              sem0, sem1):
    c = lax.axis_index("c")
    s = lax.axis_index("s")

    pltpu.sync_copy(zrow_hbm, acc_sh.at[pl.ds(s * RPT, RPT)])

    if want_cnt:
        @pl.when(c == 0)
        def _():
            pltpu.sync_copy(zcnt_hbm, cnt_sh.at[pl.ds(s * CPS, CPS)])
            pltpu.sync_copy(ones_hbm, ones_v)

    plsc.subcore_barrier()

    def start_gather(n, rows_v, sem):
        idx = srcs_v.at[n]

        @pl.when(c == 0)
        def _():
            pltpu.async_copy(ylo_hbm.at[idx], rows_v, sem)

        @pl.when(c == 1)
        def _():
            pltpu.async_copy(yhi_hbm.at[idx], rows_v, sem)

    def wait_gather(n, rows_v, sem):
        idx = srcs_v.at[n]

        @pl.when(c == 0)
        def _():
            pltpu.make_async_copy(ylo_hbm.at[idx], rows_v, sem).wait()

        @pl.when(c == 1)
        def _():
            pltpu.make_async_copy(yhi_hbm.at[idx], rows_v, sem).wait()

    def scatter(n, rows_v):
        idx = dsts_v.at[n]
        pltpu.sync_copy(rows_v, acc_sh.at[idx], add=True)

        if want_cnt:
            @pl.when(c == 0)
            def _():
                pltpu.sync_copy(ones_v, cnt_sh.at[idx], add=True)

    for ph in range(CPT // PCH):
        off = s * CPT + ph * PCH
        pltpu.sync_copy(src2_hbm.at[pl.ds(off, PCH)], srcs_v)
        pltpu.sync_copy(dst2_hbm.at[pl.ds(off, PCH)], dsts_v)
        start_gather(0, rows0_v, sem0)

        @pl.loop(0, PCH // 2)
        def _pair(k):
            n0 = 2 * k
            start_gather(n0 + 1, rows1_v, sem1)
            wait_gather(n0, rows0_v, sem0)
            scatter(n0, rows0_v)

            @pl.when(k < PCH // 2 - 1)
            def _():
                start_gather(n0 + 2, rows0_v, sem0)

            wait_gather(n0 + 1, rows1_v, sem1)
            scatter(n0 + 1, rows1_v)

    plsc.subcore_barrier()

    @pl.when(c == 0)
    def _():
        pltpu.sync_copy(acc_sh.at[pl.ds(s * RPT, RPT)], slo_hbm.at[pl.ds(s * RPT, RPT)])
        if want_cnt:
            pltpu.sync_copy(cnt_sh.at[pl.ds(s * CPS, CPS)], cnt_hbm.at[pl.ds(s * CPS, CPS)])

    @pl.when(c == 1)
    def _():
        pltpu.sync_copy(acc_sh.at[pl.ds(s * RPT, RPT)], shi_hbm.at[pl.ds(s * RPT, RPT)])


def _segment_sum_sc(ylo, yhi, src2, dst2, want_cnt):
    zrow = jnp.zeros((RPT, HH), jnp.float32)
    zcnt = jnp.zeros((CPS,), jnp.float32)
    ones = jnp.ones((EPT,), jnp.float32)
    mesh = plsc.VectorSubcoreMesh(core_axis_name="c", subcore_axis_name="s")
    return pl.kernel(
        functools.partial(_seg_body, want_cnt),
        out_type=(
            jax.ShapeDtypeStruct((NPAD, HH), jnp.float32),
            jax.ShapeDtypeStruct((NPAD, HH), jnp.float32),
            jax.ShapeDtypeStruct((NPAD,), jnp.float32),
        ),
        mesh=mesh,
        scratch_types=[
            pltpu.VMEM_SHARED((NPAD, HH), jnp.float32),
            pltpu.VMEM_SHARED((NPAD,), jnp.float32),
            pltpu.VMEM((PCH, EPT), jnp.int32),
            pltpu.VMEM((PCH, EPT), jnp.int32),
            pltpu.VMEM((EPT, HH), jnp.float32),
            pltpu.VMEM((EPT, HH), jnp.float32),
            pltpu.VMEM((EPT,), jnp.float32),
            pltpu.SemaphoreType.DMA,
            pltpu.SemaphoreType.DMA,
        ],
    )(ylo, yhi, src2, dst2, zrow, zcnt, ones)


NLBL = 100000
LPAD = 100352
NW = 32
LPW = LPAD // NW
LEPT = 112
LCH = LPW // LEPT


def _label_body(x3lo_hbm, x3hi_hbm, la_hbm, lb_hbm, pred_hbm,
                ia_v, ib_v, rA0, rB0, rA1, rB1, out_v, accb_v,
                semA0, semB0, semA1, semB1):
    c = lax.axis_index("c")
    s = lax.axis_index("s")
    wid = s * 2 + c
    base_t = wid * LPW

    pltpu.sync_copy(la_hbm.at[pl.ds(base_t, LPW)], ia_v)
    pltpu.sync_copy(lb_hbm.at[pl.ds(base_t, LPW)], ib_v)

    def start(n, h, rA, rB, sA, sB):
        tbl = x3lo_hbm if h == 0 else x3hi_hbm
        pltpu.async_copy(tbl.at[ia_v.at[pl.ds(n * LEPT, LEPT)]], rA, sA)
        pltpu.async_copy(tbl.at[ib_v.at[pl.ds(n * LEPT, LEPT)]], rB, sB)

    def wait(n, h, rA, rB, sA, sB):
        tbl = x3lo_hbm if h == 0 else x3hi_hbm
        pltpu.make_async_copy(tbl.at[ia_v.at[pl.ds(n * LEPT, LEPT)]], rA, sA).wait()
        pltpu.make_async_copy(tbl.at[ib_v.at[pl.ds(n * LEPT, LEPT)]], rB, sB).wait()

    def compute(n, h, rA, rB):
        @pl.loop(0, LEPT // 16)
        def _group(g):
            lane = lax.iota(jnp.int32, 16)
            res = jnp.zeros((16,), jnp.float32)
            for j in range(16):
                e = g * 16 + j
                acc = rA[e, pl.ds(0, 16)] * rB[e, pl.ds(0, 16)]
                for t in range(1, HH // 16):
                    acc = acc + rA[e, pl.ds(16 * t, 16)] * rB[e, pl.ds(16 * t, 16)]
                if h == 0:
                    accb_v[e, :] = acc
                else:
                    acc = acc + accb_v[e, :]
                    for sh in (8, 4, 2, 1):
                        acc = acc + acc.at[lane ^ sh].get(mode="promise_in_bounds")
                    res = jnp.where(lane == j, acc, res)
            if h == 1:
                out_v[pl.ds(n * LEPT + g * 16, 16)] = res

    start(0, 0, rA0, rB0, semA0, semB0)

    @pl.loop(0, LCH)
    def _chunk(k):
        start(k, 1, rA1, rB1, semA1, semB1)
        wait(k, 0, rA0, rB0, semA0, semB0)
        compute(k, 0, rA0, rB0)

        @pl.when(k < LCH - 1)
        def _():
            start(k + 1, 0, rA0, rB0, semA0, semB0)

        wait(k, 1, rA1, rB1, semA1, semB1)
        compute(k, 1, rA1, rB1)

    pltpu.sync_copy(out_v, pred_hbm.at[pl.ds(base_t, LPW)])


def _label_dot_sc(x3lo, x3hi, la, lb):
    mesh = plsc.VectorSubcoreMesh(core_axis_name="c", subcore_axis_name="s")
    la_p = jnp.zeros((LPAD,), la.dtype).at[:NLBL].set(la)
    lb_p = jnp.zeros((LPAD,), lb.dtype).at[:NLBL].set(lb)
    pred = pl.kernel(
        _label_body,
        out_type=jax.ShapeDtypeStruct((LPAD,), jnp.float32),
        mesh=mesh,
        scratch_types=[
            pltpu.VMEM((LPW,), jnp.int32),
            pltpu.VMEM((LPW,), jnp.int32),
            pltpu.VMEM((LEPT, HH), jnp.float32),
            pltpu.VMEM((LEPT, HH), jnp.float32),
            pltpu.VMEM((LEPT, HH), jnp.float32),
            pltpu.VMEM((LEPT, HH), jnp.float32),
            pltpu.VMEM((LPW,), jnp.float32),
            pltpu.VMEM((LEPT, 16), jnp.float32),
            pltpu.SemaphoreType.DMA,
            pltpu.SemaphoreType.DMA,
            pltpu.SemaphoreType.DMA,
            pltpu.SemaphoreType.DMA,
        ],
    )(x3lo, x3hi, la_p, lb_p)
    return pred[:NLBL]


def kernel(node_feature_n1, edge_index_n1_e1_n1, edge_label_index_n1_e1_n1,
           W_neigh1, b_neigh1, W_self1, b_self1, W_update1, b_update1,
           W_neigh2, b_neigh2, W_self2, b_self2, W_update2, b_update2):
    src = edge_index_n1_e1_n1[0].astype(jnp.int32)
    dst = edge_index_n1_e1_n1[1].astype(jnp.int32)
    src2 = jnp.zeros((EPAD,), jnp.int32).at[:E].set(src).reshape(NCHUNK, EPT)
    dst2 = jnp.full((EPAD,), PADROW, jnp.int32).at[:E].set(dst).reshape(NCHUNK, EPT)
    A1, B1, c1 = _fold(W_neigh1, b_neigh1, W_self1, b_self1, W_update1, b_update1)
    A2, B2, c2 = _fold(W_neigh2, b_neigh2, W_self2, b_self2, W_update2, b_update2)

    ylo1, yhi1, z1 = _pre(node_feature_n1, A1, B1, c1)
    slo1, shi1, cnt_pad = _segment_sum_sc(ylo1, yhi1, src2, dst2, True)
    cnt = cnt_pad.reshape(NPAD, 1)
    ylo2, yhi2, z2 = _mid(slo1, shi1, cnt, z1, A2, B2, c2)
    slo2, shi2, _ = _segment_sum_sc(ylo2, yhi2, src2, dst2, False)
    x3lo, x3hi = _post(slo2, shi2, cnt, z2)

    la = edge_label_index_n1_e1_n1[0]
    lb = edge_label_index_n1_e1_n1[1]
    return _label_dot_sc(x3lo, x3hi, la, lb)
